# Initial kernel scaffold; baseline (speedup 1.0000x reference)
import jax
import jax.numpy as jnp
from jax.experimental import pallas as pl


def kernel(primal_e_0, primal_v_0, r_head, r_tail, e_adj_index, e_adj_data, eer_adj_index, eer_adj_data, m_head2e, m_tail2v, emv_adj_index, emv_adj_data, be_L, be_R, bm_LE, bm_LV, atten_r, atten_m, gcnW1, highwayWr1, highwaybr1, gcnW2, highwayWr2, highwaybr2):
    raise NotImplementedError("write your pallas kernel here")


# speedup vs baseline: 1.0505x; 1.0505x over previous
import functools

import jax
import jax.numpy as jnp
from jax.experimental import pallas as pl
from jax.experimental.pallas import tpu as pltpu
from jax.experimental.pallas import tpu_sc as plsc

KG_E = 10000
KG_R = 1000
KG_M = 500
KG_V = 20000
D = 300
NE = 160000
ALPHA3 = 0.4
LRELU_A = 0.2


def _inv(s):
    return jnp.where(s == 0, jnp.float32(0.0), 1.0 / s)


def _norm_matmul(Wm, X, b):
    KB = 2048
    R, N = Wm.shape
    d = X.shape[1]
    pad = (-N) % KB
    if pad:
        Wm = jnp.pad(Wm, ((0, 0), (0, pad)))
        X = jnp.pad(X, ((0, pad), (0, 0)))
        b = jnp.pad(b, ((0, pad), (0, 0)))
        N += pad

    def kern(w_ref, x_ref, b_ref, o_ref, acc, rs):
        k = pl.program_id(0)

        @pl.when(k == 0)
        def _():
            acc[...] = jnp.zeros_like(acc)
            rs[...] = jnp.zeros_like(rs)

        w = w_ref[...]
        xb = x_ref[...] * b_ref[...]
        acc[...] += jnp.dot(w, xb, preferred_element_type=jnp.float32)
        rs[...] += jnp.sum(w, axis=1, keepdims=True)

        @pl.when(k == pl.num_programs(0) - 1)
        def _():
            s = rs[...]
            o_ref[...] = acc[...] * jnp.where(s == 0, 0.0, 1.0 / s)

    return pl.pallas_call(
        kern,
        grid=(N // KB,),
        in_specs=[
            pl.BlockSpec((R, KB), lambda k: (0, k)),
            pl.BlockSpec((KB, d), lambda k: (k, 0)),
            pl.BlockSpec((KB, 1), lambda k: (k, 0)),
        ],
        out_specs=pl.BlockSpec((R, d), lambda k: (0, 0)),
        out_shape=jax.ShapeDtypeStruct((R, d), jnp.float32),
        scratch_shapes=[
            pltpu.VMEM((R, d), jnp.float32),
            pltpu.VMEM((R, 1), jnp.float32),
        ],
    )(Wm, X, b)


def _matmul(X, W):
    M, K = X.shape
    Ko = W.shape[1]
    MB = 2000

    def kern(x_ref, w_ref, o_ref):
        o_ref[...] = jnp.dot(x_ref[...], w_ref[...], preferred_element_type=jnp.float32)

    return pl.pallas_call(
        kern,
        grid=(M // MB,),
        in_specs=[
            pl.BlockSpec((MB, K), lambda m: (m, 0)),
            pl.BlockSpec((K, Ko), lambda m: (0, 0)),
        ],
        out_specs=pl.BlockSpec((MB, Ko), lambda m: (m, 0)),
        out_shape=jax.ShapeDtypeStruct((M, Ko), jnp.float32),
    )(X, W)


def _highway_fuse(X, S, Wr, br):
    M, d = X.shape
    MB = 2000

    def kern(x_ref, s_ref, w_ref, b_ref, o_ref):
        x = x_ref[...]
        g = jax.nn.sigmoid(
            jnp.dot(x, w_ref[...], preferred_element_type=jnp.float32)
            + b_ref[...].reshape(1, d)
        )
        e2 = jnp.maximum(s_ref[...], 0.0)
        o_ref[...] = g * e2 + (1.0 - g) * x

    return pl.pallas_call(
        kern,
        grid=(M // MB,),
        in_specs=[
            pl.BlockSpec((MB, d), lambda m: (m, 0)),
            pl.BlockSpec((MB, d), lambda m: (m, 0)),
            pl.BlockSpec((d, d), lambda m: (0, 0)),
            pl.BlockSpec((d, 1), lambda m: (0, 0)),
        ],
        out_specs=pl.BlockSpec((MB, d), lambda m: (m, 0)),
        out_shape=jax.ShapeDtypeStruct((M, d), jnp.float32),
    )(X, S, Wr, br)


def _postatt(base, S, rs, alpha, residual):
    M, d = S.shape
    MB = 2000

    def kern(b_ref, s_ref, r_ref, o_ref):
        r = r_ref[...]
        e = jnp.maximum(s_ref[...] * jnp.where(r == 0, 0.0, 1.0 / r), 0.0)
        if residual:
            o_ref[...] = b_ref[...] + alpha * e
        else:
            o_ref[...] = e

    return pl.pallas_call(
        kern,
        grid=(M // MB,),
        in_specs=[
            pl.BlockSpec((MB, d), lambda m: (m, 0)),
            pl.BlockSpec((MB, d), lambda m: (m, 0)),
            pl.BlockSpec((MB, 1), lambda m: (m, 0)),
        ],
        out_specs=pl.BlockSpec((MB, d), lambda m: (m, 0)),
        out_shape=jax.ShapeDtypeStruct((M, d), jnp.float32),
    )(base, S, rs)


def _spmm(index, data, n_rows, mat):
    return jnp.zeros((n_rows, mat.shape[1]), mat.dtype).at[index[0]].add(
        data[:, None] * mat[index[1]]
    )


def _leaky(x):
    return jnp.where(x >= 0, x, LRELU_A * x)


def kernel(primal_e_0, primal_v_0, r_head, r_tail, e_adj_index, e_adj_data,
           eer_adj_index, eer_adj_data, m_head2e, m_tail2v, emv_adj_index,
           emv_adj_data, be_L, be_R, bm_LE, bm_LV, atten_r, atten_m, gcnW1,
           highwayWr1, highwaybr1, gcnW2, highwayWr2, highwaybr2):
    name = primal_e_0
    value = primal_v_0

    L_r = _norm_matmul(r_head, name, be_L)
    R_r = _norm_matmul(r_tail, name, be_R)
    L_m = _norm_matmul(m_head2e, name, bm_LE)
    R_m = _norm_matmul(m_tail2v, value, bm_LV)

    r_embed = jax.nn.relu(jnp.concatenate([L_r, R_r], axis=-1))
    m_embed = jax.nn.relu(jnp.concatenate([L_m, R_m], axis=-1))

    e_i = name[eer_adj_index[0]]
    e_j = name[eer_adj_index[1]]
    eer_embed = jnp.concatenate([e_i, e_j], axis=1) * r_embed[eer_adj_data]
    eer_atten = jnp.exp(-_leaky((eer_embed @ atten_r)[:, 0]))
    e_rowsum = _spmm(eer_adj_index, eer_atten, KG_E, jnp.ones((KG_E, 1), jnp.float32))
    S_att = _spmm(eer_adj_index, eer_atten, KG_E, name)
    se_embed = _postatt(name, S_att, e_rowsum, ALPHA3, True)

    e_i2 = name[emv_adj_index[0]]
    v_j = value[emv_adj_index[1]]
    emv_embed = jnp.concatenate([e_i2, v_j], axis=1) * m_embed[emv_adj_data]
    emv_atten = jnp.exp(-_leaky((emv_embed @ atten_m)[:, 0]))
    ev_rowsum = _spmm(emv_adj_index, emv_atten, KG_E, jnp.ones((KG_V, 1), jnp.float32))
    S_att2 = _spmm(emv_adj_index, emv_atten, KG_E, value)
    ce_embed = _postatt(S_att2, S_att2, ev_rowsum, 1.0, False)

    def gcn_branch(e0, gcnW, Wr, br):
        e = e0
        for _ in range(2):
            Y = _matmul(e, gcnW)
            Ssp = _spmm(e_adj_index, e_adj_data, KG_E, Y)
            e = _highway_fuse(e, Ssp, Wr, br)
        return e

    se_layer = gcn_branch(se_embed, gcnW1, highwayWr1, highwaybr1)
    ce_layer = gcn_branch(ce_embed, gcnW2, highwayWr2, highwaybr2)
    return (se_layer, ce_layer)


# speedup vs baseline: 1.4224x; 1.3540x over previous
import functools

import jax
import jax.numpy as jnp
from jax.experimental import pallas as pl
from jax.experimental.pallas import tpu as pltpu
from jax.experimental.pallas import tpu_sc as plsc

KG_E = 10000
KG_R = 1000
KG_M = 500
KG_V = 20000
D = 300
NE = 160000
ALPHA3 = 0.4
LRELU_A = 0.2


def _inv(s):
    return jnp.where(s == 0, jnp.float32(0.0), 1.0 / s)


def _norm_matmul(Wm, X, b):
    KB = 2048
    R, N = Wm.shape
    d = X.shape[1]
    pad = (-N) % KB
    if pad:
        Wm = jnp.pad(Wm, ((0, 0), (0, pad)))
        X = jnp.pad(X, ((0, pad), (0, 0)))
        b = jnp.pad(b, ((0, pad), (0, 0)))
        N += pad

    def kern(w_ref, x_ref, b_ref, o_ref, acc, rs):
        k = pl.program_id(0)

        @pl.when(k == 0)
        def _():
            acc[...] = jnp.zeros_like(acc)
            rs[...] = jnp.zeros_like(rs)

        w = w_ref[...]
        xb = x_ref[...] * b_ref[...]
        acc[...] += jnp.dot(w, xb, preferred_element_type=jnp.float32)
        rs[...] += jnp.sum(w, axis=1, keepdims=True)

        @pl.when(k == pl.num_programs(0) - 1)
        def _():
            s = rs[...]
            o_ref[...] = acc[...] * jnp.where(s == 0, 0.0, 1.0 / s)

    return pl.pallas_call(
        kern,
        grid=(N // KB,),
        in_specs=[
            pl.BlockSpec((R, KB), lambda k: (0, k)),
            pl.BlockSpec((KB, d), lambda k: (k, 0)),
            pl.BlockSpec((KB, 1), lambda k: (k, 0)),
        ],
        out_specs=pl.BlockSpec((R, d), lambda k: (0, 0)),
        out_shape=jax.ShapeDtypeStruct((R, d), jnp.float32),
        scratch_shapes=[
            pltpu.VMEM((R, d), jnp.float32),
            pltpu.VMEM((R, 1), jnp.float32),
        ],
    )(Wm, X, b)


def _matmul(X, W):
    M, K = X.shape
    Ko = W.shape[1]
    MB = 2000

    def kern(x_ref, w_ref, o_ref):
        o_ref[...] = jnp.dot(x_ref[...], w_ref[...], preferred_element_type=jnp.float32)

    return pl.pallas_call(
        kern,
        grid=(M // MB,),
        in_specs=[
            pl.BlockSpec((MB, K), lambda m: (m, 0)),
            pl.BlockSpec((K, Ko), lambda m: (0, 0)),
        ],
        out_specs=pl.BlockSpec((MB, Ko), lambda m: (m, 0)),
        out_shape=jax.ShapeDtypeStruct((M, Ko), jnp.float32),
    )(X, W)


def _highway_fuse(X, S, Wr, br):
    M, d = X.shape
    MB = 2000

    def kern(x_ref, s_ref, w_ref, b_ref, o_ref):
        x = x_ref[...]
        g = jax.nn.sigmoid(
            jnp.dot(x, w_ref[...], preferred_element_type=jnp.float32)
            + b_ref[...].reshape(1, d)
        )
        e2 = jnp.maximum(s_ref[...], 0.0)
        o_ref[...] = g * e2 + (1.0 - g) * x

    return pl.pallas_call(
        kern,
        grid=(M // MB,),
        in_specs=[
            pl.BlockSpec((MB, d), lambda m: (m, 0)),
            pl.BlockSpec((MB, d), lambda m: (m, 0)),
            pl.BlockSpec((d, d), lambda m: (0, 0)),
            pl.BlockSpec((d, 1), lambda m: (0, 0)),
        ],
        out_specs=pl.BlockSpec((MB, d), lambda m: (m, 0)),
        out_shape=jax.ShapeDtypeStruct((M, d), jnp.float32),
    )(X, S, Wr, br)


def _postatt(base, S, rs, alpha, residual):
    M, d = S.shape
    MB = 2000

    def kern(b_ref, s_ref, r_ref, o_ref):
        r = r_ref[...]
        e = jnp.maximum(s_ref[...] * jnp.where(r == 0, 0.0, 1.0 / r), 0.0)
        if residual:
            o_ref[...] = b_ref[...] + alpha * e
        else:
            o_ref[...] = e

    return pl.pallas_call(
        kern,
        grid=(M // MB,),
        in_specs=[
            pl.BlockSpec((MB, d), lambda m: (m, 0)),
            pl.BlockSpec((MB, d), lambda m: (m, 0)),
            pl.BlockSpec((MB, 1), lambda m: (m, 0)),
        ],
        out_specs=pl.BlockSpec((MB, d), lambda m: (m, 0)),
        out_shape=jax.ShapeDtypeStruct((M, d), jnp.float32),
    )(base, S, rs)


_NC = 2
_NS = 16
_L = 16
_HW = 160
_CH = 80
_N = KG_E


def _zero16(ref, n):
    z = jnp.zeros((_L,), jnp.float32)

    def bd(t, _):
        ref[pl.ds(t * _L, _L)] = z
        return _

    jax.lax.fori_loop(0, n // _L, bd, None)


def _sc_spmm(idx_i, idx_j, tab, *, att=None, w=None):
    ne = idx_i.shape[0]
    ept = ne // _NS
    cpt = ept // _CH
    is_att = att is not None
    mesh = plsc.VectorSubcoreMesh(core_axis_name="c", subcore_axis_name="s")

    out_type = [jax.ShapeDtypeStruct((2 * _N, _HW), jnp.float32)]
    if is_att:
        out_type.append(jax.ShapeDtypeStruct((_N,), jnp.float32))

    scratch = dict(
        ibuf=pltpu.VMEM((_CH,), jnp.int32),
        jbuf=pltpu.VMEM((_CH,), jnp.int32),
        wchunk=pltpu.VMEM((_CH,), jnp.float32),
        rows=pltpu.VMEM((_CH, _HW), jnp.float32),
        z1d=pltpu.VMEM((1024,), jnp.float32),
        acc=pltpu.VMEM_SHARED((_N, _HW), jnp.float32),
        sem1=pltpu.SemaphoreType.DMA,
        sem2=pltpu.SemaphoreType.DMA,
        sem3=pltpu.SemaphoreType.DMA,
        sem4=pltpu.SemaphoreType.DMA,
    )
    if is_att:
        scratch.update(
            relbuf=pltpu.VMEM((_CH,), jnp.int32),
            fibuf=pltpu.VMEM((_CH,), jnp.int32),
            fjbuf=pltpu.VMEM((_CH,), jnp.int32),
            pbuf=pltpu.VMEM((_CH,), jnp.float32),
            qbuf=pltpu.VMEM((_CH,), jnp.float32),
            rs_sh=pltpu.VMEM_SHARED((_N,), jnp.float32),
        )

    def body(*refs):
        if is_att:
            (pflat, qflat, i_h, j_h, rel_h, tab_h, out_h, rs_h, r) = (
                refs[0], refs[1], refs[2], refs[3], refs[4], refs[5],
                refs[6], refs[7], refs[8:])
        else:
            (w_h, i_h, j_h, tab_h, out_h, r) = (
                refs[0], refs[1], refs[2], refs[3], refs[4], refs[5:])
        sc = dict(zip(scratch.keys(), r))
        ibuf, jbuf, wchunk, rows = sc["ibuf"], sc["jbuf"], sc["wchunk"], sc["rows"]
        z1d, acc = sc["z1d"], sc["acc"]
        sem1, sem2, sem3, sem4 = sc["sem1"], sc["sem2"], sc["sem3"], sc["sem4"]

        c = jax.lax.axis_index("c")
        s = jax.lax.axis_index("s")

        def zrow(t, _):
            for m in range(_HW // _L):
                rows[t, pl.ds(m * _L, _L)] = jnp.zeros((_L,), jnp.float32)
            return _

        jax.lax.fori_loop(0, _CH, zrow, None)
        _zero16(z1d, 1024)

        @pl.when(s < 10)
        def _():
            for kk, nn in ((0, 400), (400, 400), (800, 200)):
                pltpu.sync_copy(
                    rows.at[pl.ds(0, nn), :],
                    acc.at[pl.ds(s * 1000 + kk, nn), :],
                )
        if is_att:
            @pl.when(jnp.logical_and(c == 0, s < 10))
            def _():
                pltpu.sync_copy(z1d.at[pl.ds(0, 1000)],
                                sc["rs_sh"].at[pl.ds(s * 1000, 1000)])
        plsc.subcore_barrier()

        base0 = s * ept

        def chunk(k, _):
            base = base0 + k * _CH
            di = pltpu.async_copy(i_h.at[pl.ds(base, _CH)], ibuf, sem1)
            dj = pltpu.async_copy(j_h.at[pl.ds(base, _CH)], jbuf, sem2)
            if is_att:
                drel = pltpu.async_copy(rel_h.at[pl.ds(base, _CH)],
                                        sc["relbuf"], sem3)
            else:
                dw = pltpu.async_copy(w_h.at[pl.ds(base, _CH)], wchunk, sem3)
            di.wait()
            dj.wait()
            if is_att:
                relbuf, fibuf, fjbuf = sc["relbuf"], sc["fibuf"], sc["fjbuf"]
                pbuf, qbuf = sc["pbuf"], sc["qbuf"]
                R = att[3]
                drel.wait()
                for m in range(_CH // _L):
                    dsl = pl.ds(m * _L, _L)
                    fibuf[dsl] = ibuf[dsl] * R + relbuf[dsl]
                    fjbuf[dsl] = jbuf[dsl] * R + relbuf[dsl]
                    jbuf[dsl] = jbuf[dsl] + c * _N
                d1 = pltpu.async_copy(pflat.at[fibuf], pbuf, sem1)
                d2 = pltpu.async_copy(qflat.at[fjbuf], qbuf, sem2)
                d3 = pltpu.async_copy(tab_h.at[jbuf], rows, sem4)
                d1.wait()
                d2.wait()
                for m in range(_CH // _L):
                    dsl = pl.ds(m * _L, _L)
                    t = pbuf[dsl] + qbuf[dsl]
                    t = jnp.where(t >= 0, t, LRELU_A * t)
                    wchunk[dsl] = jnp.exp(-t)
            else:
                for m in range(_CH // _L):
                    dsl = pl.ds(m * _L, _L)
                    jbuf[dsl] = jbuf[dsl] + c * _N
                d3 = pltpu.async_copy(tab_h.at[jbuf], rows, sem4)
                dw.wait()
            d3.wait()

            def scale16(g, _):
                w16 = wchunk[pl.ds(g * _L, _L)]

                def scale1(l, _):
                    e = g * _L + l
                    wsp = w16.at[jnp.zeros((_L,), jnp.int32) + l].get(
                        mode="promise_in_bounds")
                    for m in range(_HW // _L):
                        dsl = pl.ds(m * _L, _L)
                        rows[e, dsl] = rows[e, dsl] * wsp
                    return _

                jax.lax.fori_loop(0, _L, scale1, None)
                return _

            jax.lax.fori_loop(0, _CH // _L, scale16, None)
            pltpu.sync_copy(rows, acc.at[ibuf], add=True)
            if is_att:
                @pl.when(c == 0)
                def _():
                    pltpu.sync_copy(wchunk, sc["rs_sh"].at[ibuf], add=True)
            return _

        jax.lax.fori_loop(0, cpt, chunk, None)
        plsc.subcore_barrier()

        @pl.when(s < 10)
        def _():
            pltpu.sync_copy(acc.at[pl.ds(s * 1000, 1000), :],
                            out_h.at[pl.ds(c * _N + s * 1000, 1000), :])
        if is_att:
            @pl.when(jnp.logical_and(c == 0, s < 10))
            def _():
                pltpu.sync_copy(sc["rs_sh"].at[pl.ds(s * 1000, 1000)],
                                rs_h.at[pl.ds(s * 1000, 1000)])

    kfn = pl.kernel(
        body,
        out_type=tuple(out_type) if is_att else out_type[0],
        mesh=mesh,
        scratch_types=list(scratch.values()),
        compiler_params=pltpu.CompilerParams(use_tc_tiling_on_sc=False),
    )
    if is_att:
        return kfn(att[0], att[1], idx_i, idx_j, att[2], tab)
    return kfn(w, idx_i, idx_j, tab)


def _to_halves(X):
    return jnp.concatenate(
        [X[:, :_HW], jnp.pad(X[:, _HW:], ((0, 0), (0, 2 * _HW - D)))], axis=0
    )


def _from_halves(o):
    return jnp.concatenate([o[:_N], o[_N:, : D - _HW]], axis=1)


def _spmm(index, data, n_rows, mat):
    return jnp.zeros((n_rows, mat.shape[1]), mat.dtype).at[index[0]].add(
        data[:, None] * mat[index[1]]
    )


def _leaky(x):
    return jnp.where(x >= 0, x, LRELU_A * x)


def kernel(primal_e_0, primal_v_0, r_head, r_tail, e_adj_index, e_adj_data,
           eer_adj_index, eer_adj_data, m_head2e, m_tail2v, emv_adj_index,
           emv_adj_data, be_L, be_R, bm_LE, bm_LV, atten_r, atten_m, gcnW1,
           highwayWr1, highwaybr1, gcnW2, highwayWr2, highwaybr2):
    name = primal_e_0
    value = primal_v_0

    L_r = _norm_matmul(r_head, name, be_L)
    R_r = _norm_matmul(r_tail, name, be_R)
    L_m = _norm_matmul(m_head2e, name, bm_LE)
    R_m = _norm_matmul(m_tail2v, value, bm_LV)

    r_embed = jax.nn.relu(jnp.concatenate([L_r, R_r], axis=-1))
    m_embed = jax.nn.relu(jnp.concatenate([L_m, R_m], axis=-1))

    e_i = name[eer_adj_index[0]]
    e_j = name[eer_adj_index[1]]
    eer_embed = jnp.concatenate([e_i, e_j], axis=1) * r_embed[eer_adj_data]
    eer_atten = jnp.exp(-_leaky((eer_embed @ atten_r)[:, 0]))
    e_rowsum = _spmm(eer_adj_index, eer_atten, KG_E, jnp.ones((KG_E, 1), jnp.float32))
    S_att = _spmm(eer_adj_index, eer_atten, KG_E, name)
    se_embed = _postatt(name, S_att, e_rowsum, ALPHA3, True)

    e_i2 = name[emv_adj_index[0]]
    v_j = value[emv_adj_index[1]]
    emv_embed = jnp.concatenate([e_i2, v_j], axis=1) * m_embed[emv_adj_data]
    emv_atten = jnp.exp(-_leaky((emv_embed @ atten_m)[:, 0]))
    ev_rowsum = _spmm(emv_adj_index, emv_atten, KG_E, jnp.ones((KG_V, 1), jnp.float32))
    S_att2 = _spmm(emv_adj_index, emv_atten, KG_E, value)
    ce_embed = _postatt(S_att2, S_att2, ev_rowsum, 1.0, False)

    def gcn_branch(e0, gcnW, Wr, br):
        e = e0
        for _ in range(2):
            Y = _matmul(e, gcnW)
            o = _sc_spmm(e_adj_index[0], e_adj_index[1], _to_halves(Y),
                         w=e_adj_data)
            e = _highway_fuse(e, _from_halves(o), Wr, br)
        return e

    se_layer = gcn_branch(se_embed, gcnW1, highwayWr1, highwaybr1)
    ce_layer = gcn_branch(ce_embed, gcnW2, highwayWr2, highwaybr2)
    return (se_layer, ce_layer)


# speedup vs baseline: 4.7639x; 3.3492x over previous
import functools

import jax
import jax.numpy as jnp
from jax.experimental import pallas as pl
from jax.experimental.pallas import tpu as pltpu
from jax.experimental.pallas import tpu_sc as plsc

KG_E = 10000
KG_R = 1000
KG_M = 500
KG_V = 20000
D = 300
NE = 160000
ALPHA3 = 0.4
LRELU_A = 0.2


def _inv(s):
    return jnp.where(s == 0, jnp.float32(0.0), 1.0 / s)


def _norm_matmul(Wm, X, b):
    KB = 2048
    R, N = Wm.shape
    d = X.shape[1]
    pad = (-N) % KB
    if pad:
        Wm = jnp.pad(Wm, ((0, 0), (0, pad)))
        X = jnp.pad(X, ((0, pad), (0, 0)))
        b = jnp.pad(b, ((0, pad), (0, 0)))
        N += pad

    def kern(w_ref, x_ref, b_ref, o_ref, acc, rs):
        k = pl.program_id(0)

        @pl.when(k == 0)
        def _():
            acc[...] = jnp.zeros_like(acc)
            rs[...] = jnp.zeros_like(rs)

        w = w_ref[...]
        xb = x_ref[...] * b_ref[...]
        acc[...] += jnp.dot(w, xb, preferred_element_type=jnp.float32)
        rs[...] += jnp.sum(w, axis=1, keepdims=True)

        @pl.when(k == pl.num_programs(0) - 1)
        def _():
            s = rs[...]
            o_ref[...] = acc[...] * jnp.where(s == 0, 0.0, 1.0 / s)

    return pl.pallas_call(
        kern,
        grid=(N // KB,),
        in_specs=[
            pl.BlockSpec((R, KB), lambda k: (0, k)),
            pl.BlockSpec((KB, d), lambda k: (k, 0)),
            pl.BlockSpec((KB, 1), lambda k: (k, 0)),
        ],
        out_specs=pl.BlockSpec((R, d), lambda k: (0, 0)),
        out_shape=jax.ShapeDtypeStruct((R, d), jnp.float32),
        scratch_shapes=[
            pltpu.VMEM((R, d), jnp.float32),
            pltpu.VMEM((R, 1), jnp.float32),
        ],
    )(Wm, X, b)


def _matmul(X, W):
    M, K = X.shape
    Ko = W.shape[1]
    MB = 2000

    def kern(x_ref, w_ref, o_ref):
        o_ref[...] = jnp.dot(x_ref[...], w_ref[...], preferred_element_type=jnp.float32)

    return pl.pallas_call(
        kern,
        grid=(M // MB,),
        in_specs=[
            pl.BlockSpec((MB, K), lambda m: (m, 0)),
            pl.BlockSpec((K, Ko), lambda m: (0, 0)),
        ],
        out_specs=pl.BlockSpec((MB, Ko), lambda m: (m, 0)),
        out_shape=jax.ShapeDtypeStruct((M, Ko), jnp.float32),
    )(X, W)


def _matmul_bt(X, A, av):
    M, d = X.shape
    R = A.shape[0]
    MB = 2000

    def kern(x_ref, a_ref, v_ref, o_ref):
        b = jnp.maximum(a_ref[...], 0.0) * v_ref[...]
        o_ref[...] = jax.lax.dot_general(
            x_ref[...], b, (((1,), (1,)), ((), ())),
            preferred_element_type=jnp.float32)

    return pl.pallas_call(
        kern,
        grid=(M // MB,),
        in_specs=[
            pl.BlockSpec((MB, d), lambda m: (m, 0)),
            pl.BlockSpec((R, d), lambda m: (0, 0)),
            pl.BlockSpec((1, d), lambda m: (0, 0)),
        ],
        out_specs=pl.BlockSpec((MB, R), lambda m: (m, 0)),
        out_shape=jax.ShapeDtypeStruct((M, R), jnp.float32),
    )(X, A, av)


def _highway_fuse(X, S, Wr, br):
    M, d = X.shape
    MB = 2000

    def kern(x_ref, s_ref, w_ref, b_ref, o_ref):
        x = x_ref[...]
        g = jax.nn.sigmoid(
            jnp.dot(x, w_ref[...], preferred_element_type=jnp.float32)
            + b_ref[...].reshape(1, d)
        )
        e2 = jnp.maximum(s_ref[...], 0.0)
        o_ref[...] = g * e2 + (1.0 - g) * x

    return pl.pallas_call(
        kern,
        grid=(M // MB,),
        in_specs=[
            pl.BlockSpec((MB, d), lambda m: (m, 0)),
            pl.BlockSpec((MB, d), lambda m: (m, 0)),
            pl.BlockSpec((d, d), lambda m: (0, 0)),
            pl.BlockSpec((d, 1), lambda m: (0, 0)),
        ],
        out_specs=pl.BlockSpec((MB, d), lambda m: (m, 0)),
        out_shape=jax.ShapeDtypeStruct((M, d), jnp.float32),
    )(X, S, Wr, br)


def _postatt(base, S, rs, alpha, residual):
    M, d = S.shape
    MB = 2000

    def kern(b_ref, s_ref, r_ref, o_ref):
        r = r_ref[...]
        e = jnp.maximum(s_ref[...] * jnp.where(r == 0, 0.0, 1.0 / r), 0.0)
        if residual:
            o_ref[...] = b_ref[...] + alpha * e
        else:
            o_ref[...] = e

    return pl.pallas_call(
        kern,
        grid=(M // MB,),
        in_specs=[
            pl.BlockSpec((MB, d), lambda m: (m, 0)),
            pl.BlockSpec((MB, d), lambda m: (m, 0)),
            pl.BlockSpec((MB, 1), lambda m: (m, 0)),
        ],
        out_specs=pl.BlockSpec((MB, d), lambda m: (m, 0)),
        out_shape=jax.ShapeDtypeStruct((M, d), jnp.float32),
    )(base, S, rs)


_NC = 2
_NS = 16
_L = 16
_HW = 160
_CH = 80
_N = KG_E


def _zero16(ref, n):
    z = jnp.zeros((_L,), jnp.float32)

    def bd(t, _):
        ref[pl.ds(t * _L, _L)] = z
        return _

    jax.lax.fori_loop(0, n // _L, bd, None)


def _sc_spmm(idx_i, idx_j, tab, *, att=None, w=None):
    ne = idx_i.shape[0]
    ept = ne // _NS
    cpt = ept // _CH
    is_att = att is not None
    mesh = plsc.VectorSubcoreMesh(core_axis_name="c", subcore_axis_name="s")

    out_type = [jax.ShapeDtypeStruct((2 * _N, _HW), jnp.float32)]
    if is_att:
        out_type.append(jax.ShapeDtypeStruct((_N,), jnp.float32))

    scratch = dict(
        ibuf=pltpu.VMEM((_CH,), jnp.int32),
        jbuf=pltpu.VMEM((_CH,), jnp.int32),
        wchunk=pltpu.VMEM((_CH,), jnp.float32),
        rows=pltpu.VMEM((_CH, _HW), jnp.float32),
        z1d=pltpu.VMEM((1024,), jnp.float32),
        acc=pltpu.VMEM_SHARED((_N, _HW), jnp.float32),
        sem1=pltpu.SemaphoreType.DMA,
        sem2=pltpu.SemaphoreType.DMA,
        sem3=pltpu.SemaphoreType.DMA,
        sem4=pltpu.SemaphoreType.DMA,
    )
    if is_att:
        scratch.update(
            relbuf=pltpu.VMEM((_CH,), jnp.int32),
            fibuf=pltpu.VMEM((_CH,), jnp.int32),
            fjbuf=pltpu.VMEM((_CH,), jnp.int32),
            pbuf=pltpu.VMEM((_CH,), jnp.float32),
            qbuf=pltpu.VMEM((_CH,), jnp.float32),
            rs_sh=pltpu.VMEM_SHARED((_N,), jnp.float32),
        )

    def body(*refs):
        if is_att:
            (pflat, qflat, i_h, j_h, rel_h, tab_h, out_h, rs_h, r) = (
                refs[0], refs[1], refs[2], refs[3], refs[4], refs[5],
                refs[6], refs[7], refs[8:])
        else:
            (w_h, i_h, j_h, tab_h, out_h, r) = (
                refs[0], refs[1], refs[2], refs[3], refs[4], refs[5:])
        sc = dict(zip(scratch.keys(), r))
        ibuf, jbuf, wchunk, rows = sc["ibuf"], sc["jbuf"], sc["wchunk"], sc["rows"]
        z1d, acc = sc["z1d"], sc["acc"]
        sem1, sem2, sem3, sem4 = sc["sem1"], sc["sem2"], sc["sem3"], sc["sem4"]

        c = jax.lax.axis_index("c")
        s = jax.lax.axis_index("s")

        def zrow(t, _):
            for m in range(_HW // _L):
                rows[t, pl.ds(m * _L, _L)] = jnp.zeros((_L,), jnp.float32)
            return _

        jax.lax.fori_loop(0, _CH, zrow, None)
        _zero16(z1d, 1024)

        @pl.when(s < 10)
        def _():
            for kk in range(0, 1000, _CH):
                nn = min(_CH, 1000 - kk)
                pltpu.sync_copy(
                    rows.at[pl.ds(0, nn), :],
                    acc.at[pl.ds(s * 1000 + kk, nn), :],
                )
        if is_att:
            @pl.when(jnp.logical_and(c == 0, s < 10))
            def _():
                pltpu.sync_copy(z1d.at[pl.ds(0, 1000)],
                                sc["rs_sh"].at[pl.ds(s * 1000, 1000)])
        plsc.subcore_barrier()

        base0 = s * ept

        def chunk(k, _):
            base = base0 + k * _CH
            di = pltpu.async_copy(i_h.at[pl.ds(base, _CH)], ibuf, sem1)
            dj = pltpu.async_copy(j_h.at[pl.ds(base, _CH)], jbuf, sem2)
            if is_att:
                drel = pltpu.async_copy(rel_h.at[pl.ds(base, _CH)],
                                        sc["relbuf"], sem3)
            else:
                dw = pltpu.async_copy(w_h.at[pl.ds(base, _CH)], wchunk, sem3)
            di.wait()
            dj.wait()
            if is_att:
                relbuf, fibuf, fjbuf = sc["relbuf"], sc["fibuf"], sc["fjbuf"]
                pbuf, qbuf = sc["pbuf"], sc["qbuf"]
                R = att[3]
                drel.wait()
                for m in range(_CH // _L):
                    dsl = pl.ds(m * _L, _L)
                    fibuf[dsl] = ibuf[dsl] * R + relbuf[dsl]
                    fjbuf[dsl] = jbuf[dsl] * R + relbuf[dsl]
                    jbuf[dsl] = jbuf[dsl] + c * _N
                d1 = pltpu.async_copy(pflat.at[fibuf], pbuf, sem1)
                d2 = pltpu.async_copy(qflat.at[fjbuf], qbuf, sem2)
                d3 = pltpu.async_copy(tab_h.at[jbuf], rows, sem4)
                d1.wait()
                d2.wait()
                for m in range(_CH // _L):
                    dsl = pl.ds(m * _L, _L)
                    t = pbuf[dsl] + qbuf[dsl]
                    t = jnp.where(t >= 0, t, LRELU_A * t)
                    wchunk[dsl] = jnp.exp(-t)
            else:
                for m in range(_CH // _L):
                    dsl = pl.ds(m * _L, _L)
                    jbuf[dsl] = jbuf[dsl] + c * _N
                d3 = pltpu.async_copy(tab_h.at[jbuf], rows, sem4)
                dw.wait()
            d3.wait()

            def scale16(g, _):
                w16 = wchunk[pl.ds(g * _L, _L)]

                def scale1(l, _):
                    e = g * _L + l
                    wsp = w16.at[jnp.zeros((_L,), jnp.int32) + l].get(
                        mode="promise_in_bounds")
                    for m in range(_HW // _L):
                        dsl = pl.ds(m * _L, _L)
                        rows[e, dsl] = rows[e, dsl] * wsp
                    return _

                jax.lax.fori_loop(0, _L, scale1, None)
                return _

            jax.lax.fori_loop(0, _CH // _L, scale16, None)
            pltpu.sync_copy(rows, acc.at[ibuf], add=True)
            if is_att:
                @pl.when(c == 0)
                def _():
                    pltpu.sync_copy(wchunk, sc["rs_sh"].at[ibuf], add=True)
            return _

        jax.lax.fori_loop(0, cpt, chunk, None)
        plsc.subcore_barrier()

        @pl.when(s < 10)
        def _():
            pltpu.sync_copy(acc.at[pl.ds(s * 1000, 1000), :],
                            out_h.at[pl.ds(c * _N + s * 1000, 1000), :])
        if is_att:
            @pl.when(jnp.logical_and(c == 0, s < 10))
            def _():
                pltpu.sync_copy(sc["rs_sh"].at[pl.ds(s * 1000, 1000)],
                                rs_h.at[pl.ds(s * 1000, 1000)])

    kfn = pl.kernel(
        body,
        out_type=tuple(out_type) if is_att else out_type[0],
        mesh=mesh,
        scratch_types=list(scratch.values()),
        compiler_params=pltpu.CompilerParams(use_tc_tiling_on_sc=False),
    )
    if is_att:
        return kfn(att[0], att[1], idx_i, idx_j, att[2], tab)
    return kfn(w, idx_i, idx_j, tab)


def _to_halves(X):
    return jnp.concatenate(
        [X[:, :_HW], jnp.pad(X[:, _HW:], ((0, 0), (0, 2 * _HW - D)))], axis=0
    )


def _from_halves(o):
    return jnp.concatenate([o[:_N], o[_N:, : D - _HW]], axis=1)


def kernel(primal_e_0, primal_v_0, r_head, r_tail, e_adj_index, e_adj_data,
           eer_adj_index, eer_adj_data, m_head2e, m_tail2v, emv_adj_index,
           emv_adj_data, be_L, be_R, bm_LE, bm_LV, atten_r, atten_m, gcnW1,
           highwayWr1, highwaybr1, gcnW2, highwayWr2, highwaybr2):
    name = primal_e_0
    value = primal_v_0

    L_r = _norm_matmul(r_head, name, be_L)
    R_r = _norm_matmul(r_tail, name, be_R)
    L_m = _norm_matmul(m_head2e, name, bm_LE)
    R_m = _norm_matmul(m_tail2v, value, bm_LV)

    value10 = value[:KG_E]
    atr = atten_r.reshape(1, -1)
    atm = atten_m.reshape(1, -1)
    P = _matmul_bt(name, L_r, atr[:, :D])
    Q = _matmul_bt(name, R_r, atr[:, D:])
    P2 = _matmul_bt(name, L_m, atm[:, :D])
    Q2 = _matmul_bt(value10, R_m, atm[:, D:])

    o, rs = _sc_spmm(eer_adj_index[0], eer_adj_index[1], _to_halves(name),
                     att=(P.reshape(-1), Q.reshape(-1), eer_adj_data, KG_R))
    se_embed = _postatt(name, _from_halves(o), rs.reshape(-1, 1), ALPHA3, True)

    o2, rs2 = _sc_spmm(emv_adj_index[0], emv_adj_index[1], _to_halves(value10),
                       att=(P2.reshape(-1), Q2.reshape(-1), emv_adj_data, KG_M))
    S2 = _from_halves(o2)
    ce_embed = _postatt(S2, S2, rs2.reshape(-1, 1), 1.0, False)

    def gcn_branch(e0, gcnW, Wr, br):
        e = e0
        for _ in range(2):
            Y = _matmul(e, gcnW)
            o = _sc_spmm(e_adj_index[0], e_adj_index[1], _to_halves(Y),
                         w=e_adj_data)
            e = _highway_fuse(e, _from_halves(o), Wr, br)
        return e

    se_layer = gcn_branch(se_embed, gcnW1, highwayWr1, highwaybr1)
    ce_layer = gcn_branch(ce_embed, gcnW2, highwayWr2, highwaybr2)
    return (se_layer, ce_layer)


# speedup vs baseline: 4.7855x; 1.0045x over previous
import functools

import jax
import jax.numpy as jnp
from jax.experimental import pallas as pl
from jax.experimental.pallas import tpu as pltpu
from jax.experimental.pallas import tpu_sc as plsc

KG_E = 10000
KG_R = 1000
KG_M = 500
KG_V = 20000
D = 300
NE = 160000
ALPHA3 = 0.4
LRELU_A = 0.2


def _inv(s):
    return jnp.where(s == 0, jnp.float32(0.0), 1.0 / s)


def _norm_matmul(Wm, X, b):
    KB = 2048
    R, N = Wm.shape
    d = X.shape[1]
    pad = (-N) % KB
    if pad:
        Wm = jnp.pad(Wm, ((0, 0), (0, pad)))
        X = jnp.pad(X, ((0, pad), (0, 0)))
        b = jnp.pad(b, ((0, pad), (0, 0)))
        N += pad

    def kern(w_ref, x_ref, b_ref, o_ref, acc, rs):
        k = pl.program_id(0)

        @pl.when(k == 0)
        def _():
            acc[...] = jnp.zeros_like(acc)
            rs[...] = jnp.zeros_like(rs)

        w = w_ref[...]
        xb = x_ref[...] * b_ref[...]
        acc[...] += jnp.dot(w, xb, preferred_element_type=jnp.float32)
        rs[...] += jnp.sum(w, axis=1, keepdims=True)

        @pl.when(k == pl.num_programs(0) - 1)
        def _():
            s = rs[...]
            o_ref[...] = acc[...] * jnp.where(s == 0, 0.0, 1.0 / s)

    return pl.pallas_call(
        kern,
        grid=(N // KB,),
        in_specs=[
            pl.BlockSpec((R, KB), lambda k: (0, k)),
            pl.BlockSpec((KB, d), lambda k: (k, 0)),
            pl.BlockSpec((KB, 1), lambda k: (k, 0)),
        ],
        out_specs=pl.BlockSpec((R, d), lambda k: (0, 0)),
        out_shape=jax.ShapeDtypeStruct((R, d), jnp.float32),
        scratch_shapes=[
            pltpu.VMEM((R, d), jnp.float32),
            pltpu.VMEM((R, 1), jnp.float32),
        ],
    )(Wm, X, b)


def _matmul(X, W):
    M, K = X.shape
    Ko = W.shape[1]
    MB = 2000

    def kern(x_ref, w_ref, o_ref):
        o_ref[...] = jnp.dot(x_ref[...], w_ref[...], preferred_element_type=jnp.float32)

    return pl.pallas_call(
        kern,
        grid=(M // MB,),
        in_specs=[
            pl.BlockSpec((MB, K), lambda m: (m, 0)),
            pl.BlockSpec((K, Ko), lambda m: (0, 0)),
        ],
        out_specs=pl.BlockSpec((MB, Ko), lambda m: (m, 0)),
        out_shape=jax.ShapeDtypeStruct((M, Ko), jnp.float32),
    )(X, W)


def _matmul_bt(X, A, av):
    M, d = X.shape
    R = A.shape[0]
    MB = 2000

    def kern(x_ref, a_ref, v_ref, o_ref):
        b = jnp.maximum(a_ref[...], 0.0) * v_ref[...]
        o_ref[...] = jax.lax.dot_general(
            x_ref[...], b, (((1,), (1,)), ((), ())),
            preferred_element_type=jnp.float32)

    return pl.pallas_call(
        kern,
        grid=(M // MB,),
        in_specs=[
            pl.BlockSpec((MB, d), lambda m: (m, 0)),
            pl.BlockSpec((R, d), lambda m: (0, 0)),
            pl.BlockSpec((1, d), lambda m: (0, 0)),
        ],
        out_specs=pl.BlockSpec((MB, R), lambda m: (m, 0)),
        out_shape=jax.ShapeDtypeStruct((M, R), jnp.float32),
    )(X, A, av)


def _highway_fuse(X, S, Wr, br):
    M, d = X.shape
    MB = 2000

    def kern(x_ref, s_ref, w_ref, b_ref, o_ref):
        x = x_ref[...]
        g = jax.nn.sigmoid(
            jnp.dot(x, w_ref[...], preferred_element_type=jnp.float32)
            + b_ref[...].reshape(1, d)
        )
        e2 = jnp.maximum(s_ref[...], 0.0)
        o_ref[...] = g * e2 + (1.0 - g) * x

    return pl.pallas_call(
        kern,
        grid=(M // MB,),
        in_specs=[
            pl.BlockSpec((MB, d), lambda m: (m, 0)),
            pl.BlockSpec((MB, d), lambda m: (m, 0)),
            pl.BlockSpec((d, d), lambda m: (0, 0)),
            pl.BlockSpec((d, 1), lambda m: (0, 0)),
        ],
        out_specs=pl.BlockSpec((MB, d), lambda m: (m, 0)),
        out_shape=jax.ShapeDtypeStruct((M, d), jnp.float32),
    )(X, S, Wr, br)


def _postatt(base, S, rs, alpha, residual):
    M, d = S.shape
    MB = 2000

    def kern(b_ref, s_ref, r_ref, o_ref):
        r = r_ref[...]
        e = jnp.maximum(s_ref[...] * jnp.where(r == 0, 0.0, 1.0 / r), 0.0)
        if residual:
            o_ref[...] = b_ref[...] + alpha * e
        else:
            o_ref[...] = e

    return pl.pallas_call(
        kern,
        grid=(M // MB,),
        in_specs=[
            pl.BlockSpec((MB, d), lambda m: (m, 0)),
            pl.BlockSpec((MB, d), lambda m: (m, 0)),
            pl.BlockSpec((MB, 1), lambda m: (m, 0)),
        ],
        out_specs=pl.BlockSpec((MB, d), lambda m: (m, 0)),
        out_shape=jax.ShapeDtypeStruct((M, d), jnp.float32),
    )(base, S, rs)


_NC = 2
_NS = 16
_L = 16
_HW = 160
_CH = 80
_N = KG_E


def _zero16(ref, n):
    z = jnp.zeros((_L,), jnp.float32)

    def bd(t, _):
        ref[pl.ds(t * _L, _L)] = z
        return _

    jax.lax.fori_loop(0, n // _L, bd, None)


def _sc_spmm(idx_i, idx_j, tab, *, att=None, w=None):
    ne = idx_i.shape[0]
    ept = ne // _NS
    cpt = ept // _CH
    is_att = att is not None
    mesh = plsc.VectorSubcoreMesh(core_axis_name="c", subcore_axis_name="s")

    out_type = [jax.ShapeDtypeStruct((2 * _N, _HW), jnp.float32)]
    if is_att:
        out_type.append(jax.ShapeDtypeStruct((_N,), jnp.float32))

    scratch = dict(
        ibuf=pltpu.VMEM((_CH,), jnp.int32),
        jbuf=pltpu.VMEM((_CH,), jnp.int32),
        wchunk=pltpu.VMEM((_CH,), jnp.float32),
        rows=pltpu.VMEM((_CH, _HW), jnp.float32),
        z1d=pltpu.VMEM((1024,), jnp.float32),
        acc=pltpu.VMEM_SHARED((_N, _HW), jnp.float32),
        sem1=pltpu.SemaphoreType.DMA,
        sem2=pltpu.SemaphoreType.DMA,
        sem3=pltpu.SemaphoreType.DMA,
        sem4=pltpu.SemaphoreType.DMA,
        sem5=pltpu.SemaphoreType.DMA,
        sem6=pltpu.SemaphoreType.DMA,
    )
    if is_att:
        scratch.update(
            relbuf=pltpu.VMEM((_CH,), jnp.int32),
            fibuf=pltpu.VMEM((_CH,), jnp.int32),
            fjbuf=pltpu.VMEM((_CH,), jnp.int32),
            pbuf=pltpu.VMEM((_CH,), jnp.float32),
            qbuf=pltpu.VMEM((_CH,), jnp.float32),
            rs_sh=pltpu.VMEM_SHARED((_N,), jnp.float32),
        )

    def body(*refs):
        if is_att:
            (pflat, qflat, i_h, j_h, rel_h, tab_h, out_h, rs_h, r) = (
                refs[0], refs[1], refs[2], refs[3], refs[4], refs[5],
                refs[6], refs[7], refs[8:])
        else:
            (w_h, i_h, j_h, tab_h, out_h, r) = (
                refs[0], refs[1], refs[2], refs[3], refs[4], refs[5:])
        sc = dict(zip(scratch.keys(), r))
        ibuf, jbuf, wchunk, rows = sc["ibuf"], sc["jbuf"], sc["wchunk"], sc["rows"]
        z1d, acc = sc["z1d"], sc["acc"]
        sem1, sem2, sem3, sem4 = sc["sem1"], sc["sem2"], sc["sem3"], sc["sem4"]
        sem5, sem6 = sc["sem5"], sc["sem6"]

        c = jax.lax.axis_index("c")
        s = jax.lax.axis_index("s")

        def zrow(t, _):
            for m in range(_HW // _L):
                rows[t, pl.ds(m * _L, _L)] = jnp.zeros((_L,), jnp.float32)
            return _

        jax.lax.fori_loop(0, _CH, zrow, None)
        _zero16(z1d, 1024)

        @pl.when(s < 10)
        def _():
            for kk in range(0, 1000, _CH):
                nn = min(_CH, 1000 - kk)
                pltpu.sync_copy(
                    rows.at[pl.ds(0, nn), :],
                    acc.at[pl.ds(s * 1000 + kk, nn), :],
                )
        if is_att:
            @pl.when(jnp.logical_and(c == 0, s < 10))
            def _():
                pltpu.sync_copy(z1d.at[pl.ds(0, 1000)],
                                sc["rs_sh"].at[pl.ds(s * 1000, 1000)])
        plsc.subcore_barrier()

        base0 = s * ept

        def chunk(k, _):
            @pl.when(k > 0)
            def _():
                pltpu.make_async_copy(rows, acc.at[ibuf], sem5).wait()
                if is_att:
                    @pl.when(c == 0)
                    def _():
                        pltpu.make_async_copy(wchunk, sc["rs_sh"].at[ibuf],
                                              sem6).wait()

            base = base0 + k * _CH
            di = pltpu.async_copy(i_h.at[pl.ds(base, _CH)], ibuf, sem1)
            dj = pltpu.async_copy(j_h.at[pl.ds(base, _CH)], jbuf, sem2)
            if is_att:
                drel = pltpu.async_copy(rel_h.at[pl.ds(base, _CH)],
                                        sc["relbuf"], sem3)
            else:
                dw = pltpu.async_copy(w_h.at[pl.ds(base, _CH)], wchunk, sem3)
            di.wait()
            dj.wait()
            if is_att:
                relbuf, fibuf, fjbuf = sc["relbuf"], sc["fibuf"], sc["fjbuf"]
                pbuf, qbuf = sc["pbuf"], sc["qbuf"]
                R = att[3]
                drel.wait()
                for m in range(_CH // _L):
                    dsl = pl.ds(m * _L, _L)
                    fibuf[dsl] = ibuf[dsl] * R + relbuf[dsl]
                    fjbuf[dsl] = jbuf[dsl] * R + relbuf[dsl]
                    jbuf[dsl] = jbuf[dsl] + c * _N
                d1 = pltpu.async_copy(pflat.at[fibuf], pbuf, sem1)
                d2 = pltpu.async_copy(qflat.at[fjbuf], qbuf, sem2)
                d3 = pltpu.async_copy(tab_h.at[jbuf], rows, sem4)
                d1.wait()
                d2.wait()
                for m in range(_CH // _L):
                    dsl = pl.ds(m * _L, _L)
                    t = pbuf[dsl] + qbuf[dsl]
                    t = jnp.where(t >= 0, t, LRELU_A * t)
                    wchunk[dsl] = jnp.exp(-t)
            else:
                for m in range(_CH // _L):
                    dsl = pl.ds(m * _L, _L)
                    jbuf[dsl] = jbuf[dsl] + c * _N
                d3 = pltpu.async_copy(tab_h.at[jbuf], rows, sem4)
                dw.wait()
            d3.wait()

            def scale16(g, _):
                w16 = wchunk[pl.ds(g * _L, _L)]
                for l in range(_L):
                    e = g * _L + l
                    wsp = w16.at[jnp.full((_L,), l, jnp.int32)].get(
                        mode="promise_in_bounds")
                    for m in range(_HW // _L):
                        dsl = pl.ds(m * _L, _L)
                        rows[e, dsl] = rows[e, dsl] * wsp
                return _

            jax.lax.fori_loop(0, _CH // _L, scale16, None)
            pltpu.async_copy(rows, acc.at[ibuf], sem5, add=True)
            if is_att:
                @pl.when(c == 0)
                def _():
                    pltpu.async_copy(wchunk, sc["rs_sh"].at[ibuf], sem6,
                                     add=True)
            return _

        jax.lax.fori_loop(0, cpt, chunk, None)
        pltpu.make_async_copy(rows, acc.at[ibuf], sem5).wait()
        if is_att:
            @pl.when(c == 0)
            def _():
                pltpu.make_async_copy(wchunk, sc["rs_sh"].at[ibuf],
                                      sem6).wait()
        plsc.subcore_barrier()

        @pl.when(s < 10)
        def _():
            pltpu.sync_copy(acc.at[pl.ds(s * 1000, 1000), :],
                            out_h.at[pl.ds(c * _N + s * 1000, 1000), :])
        if is_att:
            @pl.when(jnp.logical_and(c == 0, s < 10))
            def _():
                pltpu.sync_copy(sc["rs_sh"].at[pl.ds(s * 1000, 1000)],
                                rs_h.at[pl.ds(s * 1000, 1000)])

    kfn = pl.kernel(
        body,
        out_type=tuple(out_type) if is_att else out_type[0],
        mesh=mesh,
        scratch_types=list(scratch.values()),
        compiler_params=pltpu.CompilerParams(use_tc_tiling_on_sc=False),
    )
    if is_att:
        return kfn(att[0], att[1], idx_i, idx_j, att[2], tab)
    return kfn(w, idx_i, idx_j, tab)


def _to_halves(X):
    return jnp.concatenate(
        [X[:, :_HW], jnp.pad(X[:, _HW:], ((0, 0), (0, 2 * _HW - D)))], axis=0
    )


def _from_halves(o):
    return jnp.concatenate([o[:_N], o[_N:, : D - _HW]], axis=1)


def kernel(primal_e_0, primal_v_0, r_head, r_tail, e_adj_index, e_adj_data,
           eer_adj_index, eer_adj_data, m_head2e, m_tail2v, emv_adj_index,
           emv_adj_data, be_L, be_R, bm_LE, bm_LV, atten_r, atten_m, gcnW1,
           highwayWr1, highwaybr1, gcnW2, highwayWr2, highwaybr2):
    name = primal_e_0
    value = primal_v_0

    L_r = _norm_matmul(r_head, name, be_L)
    R_r = _norm_matmul(r_tail, name, be_R)
    L_m = _norm_matmul(m_head2e, name, bm_LE)
    R_m = _norm_matmul(m_tail2v, value, bm_LV)

    value10 = value[:KG_E]
    atr = atten_r.reshape(1, -1)
    atm = atten_m.reshape(1, -1)
    P = _matmul_bt(name, L_r, atr[:, :D])
    Q = _matmul_bt(name, R_r, atr[:, D:])
    P2 = _matmul_bt(name, L_m, atm[:, :D])
    Q2 = _matmul_bt(value10, R_m, atm[:, D:])

    o, rs = _sc_spmm(eer_adj_index[0], eer_adj_index[1], _to_halves(name),
                     att=(P.reshape(-1), Q.reshape(-1), eer_adj_data, KG_R))
    se_embed = _postatt(name, _from_halves(o), rs.reshape(-1, 1), ALPHA3, True)

    o2, rs2 = _sc_spmm(emv_adj_index[0], emv_adj_index[1], _to_halves(value10),
                       att=(P2.reshape(-1), Q2.reshape(-1), emv_adj_data, KG_M))
    S2 = _from_halves(o2)
    ce_embed = _postatt(S2, S2, rs2.reshape(-1, 1), 1.0, False)

    def gcn_branch(e0, gcnW, Wr, br):
        e = e0
        for _ in range(2):
            Y = _matmul(e, gcnW)
            o = _sc_spmm(e_adj_index[0], e_adj_index[1], _to_halves(Y),
                         w=e_adj_data)
            e = _highway_fuse(e, _from_halves(o), Wr, br)
        return e

    se_layer = gcn_branch(se_embed, gcnW1, highwayWr1, highwaybr1)
    ce_layer = gcn_branch(ce_embed, gcnW2, highwayWr2, highwaybr2)
    return (se_layer, ce_layer)
